# Initial kernel scaffold; baseline (speedup 1.0000x reference)
import jax
import jax.numpy as jnp
from jax.experimental import pallas as pl


def kernel(x, pre_x, edge_index, internal_edge_index, ppi_edge_index, num_entity, name_embeddings, desc_embeddings, batch_size, ko_mask, batch_ko_masks, W_name, b_name, W_desc, b_desc, W_omic, b_omic, W_fus, b_fus, W_pre, b_pre, W_ie_root, W_ie_nbr, b_ie, W_enc_root, W_enc_nbr, b_enc, Wg1, bg1, Wg2, bg2, Wreg, breg):
    raise NotImplementedError("write your pallas kernel here")


# speedup vs baseline: 1.8270x; 1.8270x over previous
import functools

import jax
import jax.numpy as jnp
from jax import lax
from jax.experimental import pallas as pl
from jax.experimental.pallas import tpu as pltpu
from jax.experimental.pallas import tpu_sc as plsc

_F32 = jnp.float32
_EB = 128
_NSUB = 16
_NRNG = 4
_RB = 1000
_L = 16


def _lrelu(v):
    return jnp.where(v >= 0, v, 0.3 * v)


def _a1_body(ne_ref, de_ref, wn_ref, bn_ref, wd_ref, bd_ref, wf1_ref, wf2_ref,
             o_ref):
    a = _lrelu(jnp.dot(ne_ref[...], wn_ref[...],
                       preferred_element_type=_F32) + bn_ref[...])
    b = _lrelu(jnp.dot(de_ref[...], wd_ref[...],
                       preferred_element_type=_F32) + bd_ref[...])
    o_ref[...] = (jnp.dot(a, wf1_ref[...], preferred_element_type=_F32)
                  + jnp.dot(b, wf2_ref[...], preferred_element_type=_F32))


def _a2_body(x_ref, cnd_ref, ko_ref, womic_ref, bomic_ref, wf3_ref, bfus_ref,
             wn_ref, cx2_ref, h_ref):
    om = _lrelu(jnp.dot(x_ref[...], womic_ref[...],
                        preferred_element_type=_F32) + bomic_ref[...])
    cx = (cnd_ref[...] + jnp.dot(om, wf3_ref[...],
                                 preferred_element_type=_F32) + bfus_ref[...])
    cx2 = jnp.concatenate([cx, ko_ref[...]], axis=1)
    cx2_ref[...] = cx2
    h_ref[...] = jnp.dot(cx2, wn_ref[...], preferred_element_type=_F32)


def _b_body(agg_ref, deg_ref, cx2_ref, x_ref, px_ref, ko_ref,
            wroot_ref, bie_ref, wpre_ref, bpre_ref, wn2_ref,
            z_ref, h_ref):
    aggn = agg_ref[...] / jnp.maximum(deg_ref[...], 1.0)
    s1 = (jnp.dot(cx2_ref[...], wroot_ref[...], preferred_element_type=_F32)
          + aggn + bie_ref[...])
    x2 = jnp.concatenate([x_ref[...], ko_ref[...]], axis=1)
    px2 = jnp.concatenate([px_ref[...], ko_ref[...]], axis=1)
    z = (_lrelu(s1) + x2
         + jnp.dot(px2, wpre_ref[...], preferred_element_type=_F32)
         + bpre_ref[...])
    z_ref[...] = z
    h_ref[...] = jnp.dot(z, wn2_ref[...], preferred_element_type=_F32)


def _c_body(agg_ref, deg_ref, z_ref, wroot_ref, benc_ref, o_ref):
    aggn = agg_ref[...] / jnp.maximum(deg_ref[...], 1.0)
    o_ref[...] = _lrelu(
        jnp.dot(z_ref[...], wroot_ref[...], preferred_element_type=_F32)
        + aggn + benc_ref[...])


def _d_body(bz_ref, wg1_ref, bg1_ref, wg2_ref, bg2_ref, wreg_ref, breg_ref,
            o_ref, *, nb, kb):
    bz = bz_ref[...]
    t = jnp.tanh(jnp.dot(bz, wg1_ref[...], preferred_element_type=_F32)
                 + bg1_ref[...])
    s = jnp.dot(t, wg2_ref[...], preferred_element_type=_F32) + bg2_ref[...]
    rows = lax.broadcasted_iota(jnp.int32, (nb * kb, 1), 0)
    outs = []
    for b in range(nb):
        m = (rows >= b * kb) & (rows < (b + 1) * kb)
        sm = jnp.where(m, s, -jnp.inf)
        mx = jnp.max(sm)
        e = jnp.where(m, jnp.exp(s - mx), 0.0)
        w = e / jnp.sum(e)
        rd = jnp.sum(w * bz, axis=0, keepdims=True)
        outs.append(jnp.dot(rd, wreg_ref[...], preferred_element_type=_F32)
                    + breg_ref[...])
    o_ref[...] = jnp.concatenate(outs, axis=0)


def _batch_split(n_batches, sid):
    base, rem = divmod(n_batches, _NSUB)
    nb = jnp.where(sid < rem, base + 1, base)
    b0 = sid * base + jnp.minimum(sid, rem)
    return b0, nb


@functools.lru_cache(maxsize=None)
def _make_deg_sc(n_pad, n_edge):
    nb_total = n_edge // _EB
    rps = n_pad // _NSUB
    mesh = plsc.VectorSubcoreMesh(core_axis_name="c", subcore_axis_name="s")

    @functools.partial(
        pl.kernel,
        out_type=[jax.ShapeDtypeStruct((n_pad,), _F32) for _ in range(2)],
        mesh=mesh,
        scratch_types=[
            pltpu.VMEM_SHARED((n_pad,), _F32),
            pltpu.VMEM((1, _EB), jnp.int32),
            pltpu.VMEM((_EB,), _F32),
        ],
        compiler_params=pltpu.CompilerParams(use_tc_tiling_on_sc=False),
    )
    def deg(dst_a, dst_b, ones, zz, oa, ob, acc, idxd, onesv):
        cid = lax.axis_index("c")
        sid = lax.axis_index("s")
        b0, nb = _batch_split(nb_total, sid)
        r0 = pl.multiple_of(sid * rps, rps)
        pltpu.sync_copy(ones, onesv)

        def one_pass(dst, o_hbm):
            pltpu.sync_copy(zz.at[pl.ds(r0, rps)], acc.at[pl.ds(r0, rps)])
            plsc.subcore_barrier()

            def body(i, carry):
                off = pl.multiple_of((b0 + i) * _EB, _EB)
                pltpu.sync_copy(dst.at[pl.ds(off, _EB)], idxd.at[0])
                pltpu.sync_copy(onesv, acc.at[idxd.at[0]], add=True)
                return carry

            lax.fori_loop(0, nb, body, 0)
            plsc.subcore_barrier()
            pltpu.sync_copy(acc.at[pl.ds(r0, rps)], o_hbm.at[pl.ds(r0, rps)])

        @pl.when(cid == 0)
        def _():
            one_pass(dst_a, oa)

        @pl.when(cid == 1)
        def _():
            one_pass(dst_b, ob)

    return deg


@functools.lru_cache(maxsize=None)
def _make_sage_sc(n, n_edge):
    nb_total = n_edge // _EB
    rng = n // _NRNG
    acc_rows = ((rng + 1 + _NSUB - 1) // _NSUB) * _NSUB
    zps = acc_rows // _NSUB
    cp_hi = -(-rng // _NSUB)
    n_hi = rng - _NSUB * (rng // _NSUB)
    cp_lo = rng // _NSUB
    mesh = plsc.VectorSubcoreMesh(core_axis_name="c", subcore_axis_name="s")

    @functools.partial(
        pl.kernel,
        out_type=jax.ShapeDtypeStruct((n, 128), _F32),
        mesh=mesh,
        scratch_types=[
            pltpu.VMEM_SHARED((acc_rows, 128), _F32),
            pltpu.VMEM((_EB,), jnp.int32),
            pltpu.VMEM((1, _EB), jnp.int32),
            pltpu.VMEM((_EB, 128), _F32),
            pltpu.SemaphoreType.DMA,
        ],
        compiler_params=pltpu.CompilerParams(use_tc_tiling_on_sc=False),
    )
    def sage(h, esrc, edst, zz, o, acc, idxs, idxd, rows, sem):
        cid = lax.axis_index("c")
        sid = lax.axis_index("s")
        b0, nb = _batch_split(nb_total, sid)
        z0 = pl.multiple_of(sid * zps, zps)

        for rp in range(_NRNG // 2):
            lo = (cid * (_NRNG // 2) + rp) * rng
            pltpu.sync_copy(zz.at[pl.ds(z0, zps)], acc.at[pl.ds(z0, zps)])
            plsc.subcore_barrier()

            def body(i, carry):
                off = pl.multiple_of((b0 + i) * _EB, _EB)
                pltpu.sync_copy(esrc.at[pl.ds(off, _EB)], idxs)
                pltpu.sync_copy(edst.at[pl.ds(off, _EB)], idxd.at[0])
                pltpu.async_copy(h.at[idxs], rows, sem).wait()
                for v in range(_EB // _L):
                    d = idxd[0, pl.ds(v * _L, _L)] - lo
                    ok = (d >= 0) & (d < rng)
                    idxd[0, pl.ds(v * _L, _L)] = jnp.where(ok, d, rng)
                pltpu.sync_copy(rows, acc.at[idxd.at[0]], add=True)
                return carry

            lax.fori_loop(0, nb, body, 0)
            plsc.subcore_barrier()

            @pl.when(sid < n_hi)
            def _():
                s0 = sid * cp_hi
                pltpu.sync_copy(acc.at[pl.ds(s0, cp_hi)],
                                o.at[pl.ds(lo + s0, cp_hi)])

            @pl.when(sid >= n_hi)
            def _():
                s0 = n_hi * cp_hi + (sid - n_hi) * cp_lo
                pltpu.sync_copy(acc.at[pl.ds(s0, cp_lo)],
                                o.at[pl.ds(lo + s0, cp_lo)])

            plsc.subcore_barrier()

    return sage


@functools.lru_cache(maxsize=None)
def _make_gather_sc(n_rows, k_pad, d):
    mesh = plsc.VectorSubcoreMesh(core_axis_name="c", subcore_axis_name="s")

    @functools.partial(
        pl.kernel,
        out_type=jax.ShapeDtypeStruct((k_pad, d), _F32),
        mesh=mesh,
        scratch_types=[
            pltpu.VMEM((_EB,), jnp.int32),
            pltpu.VMEM((_EB, d), _F32),
            pltpu.SemaphoreType.DMA,
        ],
        compiler_params=pltpu.CompilerParams(use_tc_tiling_on_sc=False),
    )
    def gather(tab, idx, o, idxv, rows, sem):
        cid = lax.axis_index("c")
        sid = lax.axis_index("s")
        wid = sid * 2 + cid

        @pl.when(wid < k_pad // _EB)
        def _():
            off = pl.multiple_of(wid * _EB, _EB)
            pltpu.sync_copy(idx.at[pl.ds(off, _EB)], idxv)
            pltpu.async_copy(tab.at[idxv], rows, sem).wait()
            pltpu.sync_copy(rows, o.at[pl.ds(off, _EB)])

    return gather


def _full_spec(arr):
    return pl.BlockSpec(arr.shape, lambda i: tuple(0 for _ in arr.shape))


def kernel(x, pre_x, edge_index, internal_edge_index, ppi_edge_index,
           num_entity, name_embeddings, desc_embeddings, batch_size, ko_mask,
           batch_ko_masks, W_name, b_name, W_desc, b_desc, W_omic, b_omic,
           W_fus, b_fus, W_pre, b_pre, W_ie_root, W_ie_nbr, b_ie, W_enc_root,
           W_enc_nbr, b_enc, Wg1, bg1, Wg2, bg2, Wreg, breg):
    n, f = x.shape
    ne = name_embeddings.shape[0]
    b_sz = batch_ko_masks.shape[0]
    kb = batch_ko_masks.shape[1]
    n_pad = ((n + 16 * _NSUB - 1) // (16 * _NSUB)) * (16 * _NSUB)
    e_int = internal_edge_index.astype(jnp.int32)
    e_ext = edge_index.astype(jnp.int32)
    n_edge = e_int.shape[1]

    ko = jnp.zeros((n, 1), _F32).at[ko_mask].set(1.0)
    wf1 = W_fus[0:128]
    wf2 = W_fus[128:256]
    wf3 = W_fus[256:383]
    r2 = lambda v: v.reshape(1, -1)

    ones_eb = jnp.ones((_EB,), _F32)
    zeros_1d = jnp.zeros((n_pad,), _F32)
    deg_int, deg_ext = _make_deg_sc(n_pad, n_edge)(
        e_int[1], e_ext[1], ones_eb, zeros_1d)
    deg_int2 = deg_int[:n].reshape(n, 1)
    deg_ext2 = deg_ext[:n].reshape(n, 1)

    grid1 = ne // _RB
    c_nd = pl.pallas_call(
        _a1_body,
        grid=(grid1,),
        in_specs=[
            pl.BlockSpec((_RB, 128), lambda i: (i, 0)),
            pl.BlockSpec((_RB, 128), lambda i: (i, 0)),
        ] + [_full_spec(a) for a in
             (W_name, r2(b_name), W_desc, r2(b_desc), wf1, wf2)],
        out_specs=pl.BlockSpec((_RB, f), lambda i: (i, 0)),
        out_shape=jax.ShapeDtypeStruct((ne, f), _F32),
    )(name_embeddings, desc_embeddings, W_name, r2(b_name), W_desc,
      r2(b_desc), wf1, wf2)

    grid2 = n // _RB
    nd_blocks = ne // _RB
    cx2, h1 = pl.pallas_call(
        _a2_body,
        grid=(grid2,),
        in_specs=[
            pl.BlockSpec((_RB, f), lambda i: (i, 0)),
            pl.BlockSpec((_RB, f), lambda i: (i % nd_blocks, 0)),
            pl.BlockSpec((_RB, 1), lambda i: (i, 0)),
        ] + [_full_spec(a) for a in
             (W_omic, r2(b_omic), wf3, r2(b_fus), W_ie_nbr)],
        out_specs=[pl.BlockSpec((_RB, 128), lambda i: (i, 0))] * 2,
        out_shape=[jax.ShapeDtypeStruct((n, 128), _F32)] * 2,
    )(x, c_nd, ko, W_omic, r2(b_omic), wf3, r2(b_fus), W_ie_nbr)

    sage = _make_sage_sc(n, n_edge)
    acc_rows = ((n // _NRNG + _NSUB) // _NSUB) * _NSUB
    zeros_acc = jnp.zeros((acc_rows, 128), _F32)
    agg1 = sage(h1, e_int[0], e_int[1], zeros_acc)

    z, h2 = pl.pallas_call(
        _b_body,
        grid=(grid2,),
        in_specs=[
            pl.BlockSpec((_RB, 128), lambda i: (i, 0)),
            pl.BlockSpec((_RB, 1), lambda i: (i, 0)),
            pl.BlockSpec((_RB, 128), lambda i: (i, 0)),
            pl.BlockSpec((_RB, f), lambda i: (i, 0)),
            pl.BlockSpec((_RB, f), lambda i: (i, 0)),
            pl.BlockSpec((_RB, 1), lambda i: (i, 0)),
        ] + [_full_spec(a) for a in
             (W_ie_root, r2(b_ie), W_pre, r2(b_pre), W_enc_nbr)],
        out_specs=[pl.BlockSpec((_RB, 128), lambda i: (i, 0))] * 2,
        out_shape=[jax.ShapeDtypeStruct((n, 128), _F32)] * 2,
    )(agg1, deg_int2, cx2, x, pre_x, ko, W_ie_root, r2(b_ie),
      W_pre, r2(b_pre), W_enc_nbr)

    agg2 = sage(h2, e_ext[0], e_ext[1], zeros_acc)

    z2 = pl.pallas_call(
        _c_body,
        grid=(grid2,),
        in_specs=[
            pl.BlockSpec((_RB, 128), lambda i: (i, 0)),
            pl.BlockSpec((_RB, 1), lambda i: (i, 0)),
            pl.BlockSpec((_RB, 128), lambda i: (i, 0)),
        ] + [_full_spec(a) for a in (W_enc_root, r2(b_enc))],
        out_specs=pl.BlockSpec((_RB, 128), lambda i: (i, 0)),
        out_shape=jax.ShapeDtypeStruct((n, 128), _F32),
    )(agg2, deg_ext2, z, W_enc_root, r2(b_enc))

    bb = jnp.minimum(jnp.arange(b_sz), batch_size - 1)
    idx = (batch_ko_masks[bb].astype(jnp.int32)
           + (bb * num_entity).astype(jnp.int32)[:, None]).reshape(-1)
    k_tot = b_sz * kb
    k_pad = ((k_tot + _EB - 1) // _EB) * _EB
    idx = jnp.pad(idx, (0, k_pad - k_tot))
    bz = _make_gather_sc(n, k_pad, 128)(z2, idx)[:k_tot]

    outd = pl.pallas_call(
        functools.partial(_d_body, nb=b_sz, kb=kb),
        out_shape=jax.ShapeDtypeStruct((b_sz, 1), _F32),
    )(bz, Wg1, r2(bg1), Wg2, r2(bg2), Wreg, r2(breg))
    return outd[:, 0]


# speedup vs baseline: 1.8303x; 1.0018x over previous
import functools

import jax
import jax.numpy as jnp
from jax import lax
from jax.experimental import pallas as pl
from jax.experimental.pallas import tpu as pltpu
from jax.experimental.pallas import tpu_sc as plsc

_F32 = jnp.float32
_EB = 128
_NSUB = 16
_NRNG = 4
_RB = 1000
_L = 16


def _lrelu(v):
    return jnp.where(v >= 0, v, 0.3 * v)


def _a1_body(ne_ref, de_ref, wn_ref, bn_ref, wd_ref, bd_ref, wf1_ref, wf2_ref,
             o_ref):
    a = _lrelu(jnp.dot(ne_ref[...], wn_ref[...],
                       preferred_element_type=_F32) + bn_ref[...])
    b = _lrelu(jnp.dot(de_ref[...], wd_ref[...],
                       preferred_element_type=_F32) + bd_ref[...])
    o_ref[...] = (jnp.dot(a, wf1_ref[...], preferred_element_type=_F32)
                  + jnp.dot(b, wf2_ref[...], preferred_element_type=_F32))


def _a2_body(x_ref, cnd_ref, ko_ref, womic_ref, bomic_ref, wf3_ref, bfus_ref,
             wn_ref, cx2_ref, h_ref):
    om = _lrelu(jnp.dot(x_ref[...], womic_ref[...],
                        preferred_element_type=_F32) + bomic_ref[...])
    cx = (cnd_ref[...] + jnp.dot(om, wf3_ref[...],
                                 preferred_element_type=_F32) + bfus_ref[...])
    cx2 = jnp.concatenate([cx, ko_ref[...]], axis=1)
    cx2_ref[...] = cx2
    h_ref[...] = jnp.dot(cx2, wn_ref[...], preferred_element_type=_F32)


def _b_body(agg_ref, deg_ref, cx2_ref, x_ref, px_ref, ko_ref,
            wroot_ref, bie_ref, wpre_ref, bpre_ref, wn2_ref,
            z_ref, h_ref):
    aggn = agg_ref[...] / jnp.maximum(deg_ref[...], 1.0)
    s1 = (jnp.dot(cx2_ref[...], wroot_ref[...], preferred_element_type=_F32)
          + aggn + bie_ref[...])
    x2 = jnp.concatenate([x_ref[...], ko_ref[...]], axis=1)
    px2 = jnp.concatenate([px_ref[...], ko_ref[...]], axis=1)
    z = (_lrelu(s1) + x2
         + jnp.dot(px2, wpre_ref[...], preferred_element_type=_F32)
         + bpre_ref[...])
    z_ref[...] = z
    h_ref[...] = jnp.dot(z, wn2_ref[...], preferred_element_type=_F32)


def _c_body(agg_ref, deg_ref, z_ref, wroot_ref, benc_ref, o_ref):
    aggn = agg_ref[...] / jnp.maximum(deg_ref[...], 1.0)
    o_ref[...] = _lrelu(
        jnp.dot(z_ref[...], wroot_ref[...], preferred_element_type=_F32)
        + aggn + benc_ref[...])


def _d_body(bz_ref, wg1_ref, bg1_ref, wg2_ref, bg2_ref, wreg_ref, breg_ref,
            o_ref, *, nb, kb):
    bz = bz_ref[...]
    t = jnp.tanh(jnp.dot(bz, wg1_ref[...], preferred_element_type=_F32)
                 + bg1_ref[...])
    s = jnp.dot(t, wg2_ref[...], preferred_element_type=_F32) + bg2_ref[...]
    rows = lax.broadcasted_iota(jnp.int32, (nb * kb, 1), 0)
    outs = []
    for b in range(nb):
        m = (rows >= b * kb) & (rows < (b + 1) * kb)
        sm = jnp.where(m, s, -jnp.inf)
        mx = jnp.max(sm)
        e = jnp.where(m, jnp.exp(s - mx), 0.0)
        w = e / jnp.sum(e)
        rd = jnp.sum(w * bz, axis=0, keepdims=True)
        outs.append(jnp.dot(rd, wreg_ref[...], preferred_element_type=_F32)
                    + breg_ref[...])
    o_ref[...] = jnp.concatenate(outs, axis=0)


def _batch_split(n_batches, sid):
    base, rem = divmod(n_batches, _NSUB)
    nb = jnp.where(sid < rem, base + 1, base)
    b0 = sid * base + jnp.minimum(sid, rem)
    return b0, nb


@functools.lru_cache(maxsize=None)
def _make_deg_sc(n_pad, n_edge):
    nb_total = n_edge // _EB
    rps = n_pad // _NSUB
    mesh = plsc.VectorSubcoreMesh(core_axis_name="c", subcore_axis_name="s")

    @functools.partial(
        pl.kernel,
        out_type=[jax.ShapeDtypeStruct((n_pad,), _F32) for _ in range(2)],
        mesh=mesh,
        scratch_types=[
            pltpu.VMEM_SHARED((n_pad,), _F32),
            pltpu.VMEM((1, _EB), jnp.int32),
            pltpu.VMEM((_EB,), _F32),
        ],
        compiler_params=pltpu.CompilerParams(use_tc_tiling_on_sc=False),
    )
    def deg(dst_a, dst_b, ones, zz, oa, ob, acc, idxd, onesv):
        cid = lax.axis_index("c")
        sid = lax.axis_index("s")
        b0, nb = _batch_split(nb_total, sid)
        r0 = pl.multiple_of(sid * rps, rps)
        pltpu.sync_copy(ones, onesv)

        def one_pass(dst, o_hbm):
            pltpu.sync_copy(zz.at[pl.ds(r0, rps)], acc.at[pl.ds(r0, rps)])
            plsc.subcore_barrier()

            def body(i, carry):
                off = pl.multiple_of((b0 + i) * _EB, _EB)
                pltpu.sync_copy(dst.at[pl.ds(off, _EB)], idxd.at[0])
                pltpu.sync_copy(onesv, acc.at[idxd.at[0]], add=True)
                return carry

            lax.fori_loop(0, nb, body, 0)
            plsc.subcore_barrier()
            pltpu.sync_copy(acc.at[pl.ds(r0, rps)], o_hbm.at[pl.ds(r0, rps)])

        @pl.when(cid == 0)
        def _():
            one_pass(dst_a, oa)

        @pl.when(cid == 1)
        def _():
            one_pass(dst_b, ob)

    return deg


@functools.lru_cache(maxsize=None)
def _make_sage_sc(n, n_edge):
    ebs = 80
    nb_total = n_edge // ebs
    rng = n // _NRNG
    acc_rows = ((rng + 1 + _NSUB - 1) // _NSUB) * _NSUB
    zps = acc_rows // _NSUB
    cp_hi = -(-rng // _NSUB)
    n_hi = rng - _NSUB * (rng // _NSUB)
    cp_lo = rng // _NSUB
    mesh = plsc.VectorSubcoreMesh(core_axis_name="c", subcore_axis_name="s")

    nb_lo = (nb_total // _NSUB) & ~1
    nb_hi = nb_total - (_NSUB - 1) * nb_lo

    @functools.partial(
        pl.kernel,
        out_type=jax.ShapeDtypeStruct((n, 128), _F32),
        mesh=mesh,
        scratch_types=[
            pltpu.VMEM_SHARED((acc_rows, 128), _F32),
            pltpu.VMEM((2, ebs), jnp.int32),
            pltpu.VMEM((2, ebs), jnp.int32),
            pltpu.VMEM((2, ebs, 128), _F32),
            pltpu.SemaphoreType.DMA,
            pltpu.SemaphoreType.DMA,
        ],
        compiler_params=pltpu.CompilerParams(use_tc_tiling_on_sc=False),
    )
    def sage(h, esrc, edst, zz, o, acc, idxs, idxd, rows, sem_g, sem_s):
        cid = lax.axis_index("c")
        sid = lax.axis_index("s")
        nb = jnp.where(sid < _NSUB - 1, nb_lo, nb_hi)
        b0 = sid * nb_lo
        z0 = pl.multiple_of(sid * zps, zps)

        for rp in range(_NRNG // 2):
            lo = (cid * (_NRNG // 2) + rp) * rng
            pltpu.sync_copy(zz.at[pl.ds(z0, zps)], acc.at[pl.ds(z0, zps)])
            plsc.subcore_barrier()

            def body(j, carry):
                gd = []
                for b in range(2):
                    off = pl.multiple_of((b0 + 2 * j + b) * ebs, ebs)
                    pltpu.sync_copy(esrc.at[pl.ds(off, ebs)], idxs.at[b])
                    pltpu.sync_copy(edst.at[pl.ds(off, ebs)], idxd.at[b])
                    gd.append(pltpu.async_copy(h.at[idxs.at[b]], rows.at[b],
                                               sem_g))
                for b in range(2):
                    gd[b].wait()
                for b in range(2):
                    for v in range(ebs // _L):
                        d = idxd[b, pl.ds(v * _L, _L)] - lo
                        ok = (d >= 0) & (d < rng)
                        idxd[b, pl.ds(v * _L, _L)] = jnp.where(ok, d, rng)
                sd = [pltpu.async_copy(rows.at[b], acc.at[idxd.at[b]], sem_s,
                                       add=True) for b in range(2)]
                for b in range(2):
                    sd[b].wait()
                return carry

            lax.fori_loop(0, nb // 2, body, 0)
            plsc.subcore_barrier()

            @pl.when(sid < n_hi)
            def _():
                s0 = sid * cp_hi
                pltpu.sync_copy(acc.at[pl.ds(s0, cp_hi)],
                                o.at[pl.ds(lo + s0, cp_hi)])

            @pl.when(sid >= n_hi)
            def _():
                s0 = n_hi * cp_hi + (sid - n_hi) * cp_lo
                pltpu.sync_copy(acc.at[pl.ds(s0, cp_lo)],
                                o.at[pl.ds(lo + s0, cp_lo)])

            plsc.subcore_barrier()

    return sage


@functools.lru_cache(maxsize=None)
def _make_gather_sc(n_rows, k_pad, d):
    mesh = plsc.VectorSubcoreMesh(core_axis_name="c", subcore_axis_name="s")

    @functools.partial(
        pl.kernel,
        out_type=jax.ShapeDtypeStruct((k_pad, d), _F32),
        mesh=mesh,
        scratch_types=[
            pltpu.VMEM((_EB,), jnp.int32),
            pltpu.VMEM((_EB, d), _F32),
            pltpu.SemaphoreType.DMA,
        ],
        compiler_params=pltpu.CompilerParams(use_tc_tiling_on_sc=False),
    )
    def gather(tab, idx, o, idxv, rows, sem):
        cid = lax.axis_index("c")
        sid = lax.axis_index("s")
        wid = sid * 2 + cid

        @pl.when(wid < k_pad // _EB)
        def _():
            off = pl.multiple_of(wid * _EB, _EB)
            pltpu.sync_copy(idx.at[pl.ds(off, _EB)], idxv)
            pltpu.async_copy(tab.at[idxv], rows, sem).wait()
            pltpu.sync_copy(rows, o.at[pl.ds(off, _EB)])

    return gather


def _full_spec(arr):
    return pl.BlockSpec(arr.shape, lambda i: tuple(0 for _ in arr.shape))


def kernel(x, pre_x, edge_index, internal_edge_index, ppi_edge_index,
           num_entity, name_embeddings, desc_embeddings, batch_size, ko_mask,
           batch_ko_masks, W_name, b_name, W_desc, b_desc, W_omic, b_omic,
           W_fus, b_fus, W_pre, b_pre, W_ie_root, W_ie_nbr, b_ie, W_enc_root,
           W_enc_nbr, b_enc, Wg1, bg1, Wg2, bg2, Wreg, breg):
    n, f = x.shape
    ne = name_embeddings.shape[0]
    b_sz = batch_ko_masks.shape[0]
    kb = batch_ko_masks.shape[1]
    n_pad = ((n + 16 * _NSUB - 1) // (16 * _NSUB)) * (16 * _NSUB)
    e_int = internal_edge_index.astype(jnp.int32)
    e_ext = edge_index.astype(jnp.int32)
    n_edge = e_int.shape[1]

    ko = jnp.zeros((n, 1), _F32).at[ko_mask].set(1.0)
    wf1 = W_fus[0:128]
    wf2 = W_fus[128:256]
    wf3 = W_fus[256:383]
    r2 = lambda v: v.reshape(1, -1)

    ones_eb = jnp.ones((_EB,), _F32)
    zeros_1d = jnp.zeros((n_pad,), _F32)
    deg_int, deg_ext = _make_deg_sc(n_pad, n_edge)(
        e_int[1], e_ext[1], ones_eb, zeros_1d)
    deg_int2 = deg_int[:n].reshape(n, 1)
    deg_ext2 = deg_ext[:n].reshape(n, 1)

    grid1 = ne // _RB
    c_nd = pl.pallas_call(
        _a1_body,
        grid=(grid1,),
        in_specs=[
            pl.BlockSpec((_RB, 128), lambda i: (i, 0)),
            pl.BlockSpec((_RB, 128), lambda i: (i, 0)),
        ] + [_full_spec(a) for a in
             (W_name, r2(b_name), W_desc, r2(b_desc), wf1, wf2)],
        out_specs=pl.BlockSpec((_RB, f), lambda i: (i, 0)),
        out_shape=jax.ShapeDtypeStruct((ne, f), _F32),
    )(name_embeddings, desc_embeddings, W_name, r2(b_name), W_desc,
      r2(b_desc), wf1, wf2)

    grid2 = n // _RB
    nd_blocks = ne // _RB
    cx2, h1 = pl.pallas_call(
        _a2_body,
        grid=(grid2,),
        in_specs=[
            pl.BlockSpec((_RB, f), lambda i: (i, 0)),
            pl.BlockSpec((_RB, f), lambda i: (i % nd_blocks, 0)),
            pl.BlockSpec((_RB, 1), lambda i: (i, 0)),
        ] + [_full_spec(a) for a in
             (W_omic, r2(b_omic), wf3, r2(b_fus), W_ie_nbr)],
        out_specs=[pl.BlockSpec((_RB, 128), lambda i: (i, 0))] * 2,
        out_shape=[jax.ShapeDtypeStruct((n, 128), _F32)] * 2,
    )(x, c_nd, ko, W_omic, r2(b_omic), wf3, r2(b_fus), W_ie_nbr)

    sage = _make_sage_sc(n, n_edge)
    acc_rows = ((n // _NRNG + _NSUB) // _NSUB) * _NSUB
    zeros_acc = jnp.zeros((acc_rows, 128), _F32)
    agg1 = sage(h1, e_int[0], e_int[1], zeros_acc)

    z, h2 = pl.pallas_call(
        _b_body,
        grid=(grid2,),
        in_specs=[
            pl.BlockSpec((_RB, 128), lambda i: (i, 0)),
            pl.BlockSpec((_RB, 1), lambda i: (i, 0)),
            pl.BlockSpec((_RB, 128), lambda i: (i, 0)),
            pl.BlockSpec((_RB, f), lambda i: (i, 0)),
            pl.BlockSpec((_RB, f), lambda i: (i, 0)),
            pl.BlockSpec((_RB, 1), lambda i: (i, 0)),
        ] + [_full_spec(a) for a in
             (W_ie_root, r2(b_ie), W_pre, r2(b_pre), W_enc_nbr)],
        out_specs=[pl.BlockSpec((_RB, 128), lambda i: (i, 0))] * 2,
        out_shape=[jax.ShapeDtypeStruct((n, 128), _F32)] * 2,
    )(agg1, deg_int2, cx2, x, pre_x, ko, W_ie_root, r2(b_ie),
      W_pre, r2(b_pre), W_enc_nbr)

    agg2 = sage(h2, e_ext[0], e_ext[1], zeros_acc)

    z2 = pl.pallas_call(
        _c_body,
        grid=(grid2,),
        in_specs=[
            pl.BlockSpec((_RB, 128), lambda i: (i, 0)),
            pl.BlockSpec((_RB, 1), lambda i: (i, 0)),
            pl.BlockSpec((_RB, 128), lambda i: (i, 0)),
        ] + [_full_spec(a) for a in (W_enc_root, r2(b_enc))],
        out_specs=pl.BlockSpec((_RB, 128), lambda i: (i, 0)),
        out_shape=jax.ShapeDtypeStruct((n, 128), _F32),
    )(agg2, deg_ext2, z, W_enc_root, r2(b_enc))

    bb = jnp.minimum(jnp.arange(b_sz), batch_size - 1)
    idx = (batch_ko_masks[bb].astype(jnp.int32)
           + (bb * num_entity).astype(jnp.int32)[:, None]).reshape(-1)
    k_tot = b_sz * kb
    k_pad = ((k_tot + _EB - 1) // _EB) * _EB
    idx = jnp.pad(idx, (0, k_pad - k_tot))
    bz = _make_gather_sc(n, k_pad, 128)(z2, idx)[:k_tot]

    outd = pl.pallas_call(
        functools.partial(_d_body, nb=b_sz, kb=kb),
        out_shape=jax.ShapeDtypeStruct((b_sz, 1), _F32),
    )(bz, Wg1, r2(bg1), Wg2, r2(bg2), Wreg, r2(breg))
    return outd[:, 0]


# speedup vs baseline: 2.0979x; 1.1462x over previous
import functools

import jax
import jax.numpy as jnp
from jax import lax
from jax.experimental import pallas as pl
from jax.experimental.pallas import tpu as pltpu
from jax.experimental.pallas import tpu_sc as plsc

_F32 = jnp.float32
_EB = 128
_NSUB = 16
_NRNG = 4
_RB = 1000
_L = 16


def _lrelu(v):
    return jnp.where(v >= 0, v, 0.3 * v)


def _a1_body(ne_ref, de_ref, wn_ref, bn_ref, wd_ref, bd_ref, wf1_ref, wf2_ref,
             o_ref):
    a = _lrelu(jnp.dot(ne_ref[...], wn_ref[...],
                       preferred_element_type=_F32) + bn_ref[...])
    b = _lrelu(jnp.dot(de_ref[...], wd_ref[...],
                       preferred_element_type=_F32) + bd_ref[...])
    o_ref[...] = (jnp.dot(a, wf1_ref[...], preferred_element_type=_F32)
                  + jnp.dot(b, wf2_ref[...], preferred_element_type=_F32))


def _a2_body(x_ref, cnd_ref, ko_ref, womic_ref, bomic_ref, wf3_ref, bfus_ref,
             wn_ref, cx2_ref, h_ref):
    om = _lrelu(jnp.dot(x_ref[...], womic_ref[...],
                        preferred_element_type=_F32) + bomic_ref[...])
    cx = (cnd_ref[...] + jnp.dot(om, wf3_ref[...],
                                 preferred_element_type=_F32) + bfus_ref[...])
    cx2 = jnp.concatenate([cx, ko_ref[...]], axis=1)
    cx2_ref[...] = cx2
    h_ref[...] = jnp.dot(cx2, wn_ref[...], preferred_element_type=_F32)


def _b_body(agg_ref, deg_ref, cx2_ref, x_ref, px_ref, ko_ref,
            wroot_ref, bie_ref, wpre_ref, bpre_ref, wn2_ref,
            z_ref, h_ref):
    aggn = agg_ref[...] / jnp.maximum(deg_ref[...], 1.0)
    s1 = (jnp.dot(cx2_ref[...], wroot_ref[...], preferred_element_type=_F32)
          + aggn + bie_ref[...])
    x2 = jnp.concatenate([x_ref[...], ko_ref[...]], axis=1)
    px2 = jnp.concatenate([px_ref[...], ko_ref[...]], axis=1)
    z = (_lrelu(s1) + x2
         + jnp.dot(px2, wpre_ref[...], preferred_element_type=_F32)
         + bpre_ref[...])
    z_ref[...] = z
    h_ref[...] = jnp.dot(z, wn2_ref[...], preferred_element_type=_F32)


def _c_body(agg_ref, deg_ref, z_ref, wroot_ref, benc_ref, o_ref):
    aggn = agg_ref[...] / jnp.maximum(deg_ref[...], 1.0)
    o_ref[...] = _lrelu(
        jnp.dot(z_ref[...], wroot_ref[...], preferred_element_type=_F32)
        + aggn + benc_ref[...])


def _d_body(bz_ref, wg1_ref, bg1_ref, wg2_ref, bg2_ref, wreg_ref, breg_ref,
            o_ref, *, nb, kb):
    bz = bz_ref[...]
    t = jnp.tanh(jnp.dot(bz, wg1_ref[...], preferred_element_type=_F32)
                 + bg1_ref[...])
    s = jnp.dot(t, wg2_ref[...], preferred_element_type=_F32) + bg2_ref[...]
    rows = lax.broadcasted_iota(jnp.int32, (nb * kb, 1), 0)
    outs = []
    for b in range(nb):
        m = (rows >= b * kb) & (rows < (b + 1) * kb)
        sm = jnp.where(m, s, -jnp.inf)
        mx = jnp.max(sm)
        e = jnp.where(m, jnp.exp(s - mx), 0.0)
        w = e / jnp.sum(e)
        rd = jnp.sum(w * bz, axis=0, keepdims=True)
        outs.append(jnp.dot(rd, wreg_ref[...], preferred_element_type=_F32)
                    + breg_ref[...])
    o_ref[...] = jnp.concatenate(outs, axis=0)


def _batch_split(n_batches, sid):
    base, rem = divmod(n_batches, _NSUB)
    nb = jnp.where(sid < rem, base + 1, base)
    b0 = sid * base + jnp.minimum(sid, rem)
    return b0, nb


@functools.lru_cache(maxsize=None)
def _make_deg_sc(n_pad, n_edge):
    nb_total = n_edge // _EB
    rps = n_pad // _NSUB
    mesh = plsc.VectorSubcoreMesh(core_axis_name="c", subcore_axis_name="s")

    @functools.partial(
        pl.kernel,
        out_type=[jax.ShapeDtypeStruct((n_pad,), _F32) for _ in range(2)],
        mesh=mesh,
        scratch_types=[
            pltpu.VMEM_SHARED((n_pad,), _F32),
            pltpu.VMEM((1, _EB), jnp.int32),
            pltpu.VMEM((_EB,), _F32),
        ],
        compiler_params=pltpu.CompilerParams(use_tc_tiling_on_sc=False),
    )
    def deg(dst_a, dst_b, ones, zz, oa, ob, acc, idxd, onesv):
        cid = lax.axis_index("c")
        sid = lax.axis_index("s")
        b0, nb = _batch_split(nb_total, sid)
        r0 = pl.multiple_of(sid * rps, rps)
        pltpu.sync_copy(ones, onesv)

        def one_pass(dst, o_hbm):
            pltpu.sync_copy(zz.at[pl.ds(r0, rps)], acc.at[pl.ds(r0, rps)])
            plsc.subcore_barrier()

            def body(i, carry):
                off = pl.multiple_of((b0 + i) * _EB, _EB)
                pltpu.sync_copy(dst.at[pl.ds(off, _EB)], idxd.at[0])
                pltpu.sync_copy(onesv, acc.at[idxd.at[0]], add=True)
                return carry

            lax.fori_loop(0, nb, body, 0)
            plsc.subcore_barrier()
            pltpu.sync_copy(acc.at[pl.ds(r0, rps)], o_hbm.at[pl.ds(r0, rps)])

        @pl.when(cid == 0)
        def _():
            one_pass(dst_a, oa)

        @pl.when(cid == 1)
        def _():
            one_pass(dst_b, ob)

    return deg


@functools.lru_cache(maxsize=None)
def _make_sage_sc(n, n_edge):
    ebs = 80
    nb_total = n_edge // ebs
    rng = n // _NRNG
    acc_rows = ((rng + 1 + _NSUB - 1) // _NSUB) * _NSUB
    zps = acc_rows // _NSUB
    cp_hi = -(-rng // _NSUB)
    n_hi = rng - _NSUB * (rng // _NSUB)
    cp_lo = rng // _NSUB
    mesh = plsc.VectorSubcoreMesh(core_axis_name="c", subcore_axis_name="s")

    scan = 2000
    ebs = 80
    eps = n_edge // _NSUB
    nscan = eps // scan
    nbat = scan // ebs

    @functools.partial(
        pl.kernel,
        out_type=jax.ShapeDtypeStruct((n, 128), _F32),
        mesh=mesh,
        scratch_types=[
            pltpu.VMEM_SHARED((acc_rows, 128), _F32),
            pltpu.VMEM((scan,), jnp.int32),
            pltpu.VMEM((scan,), jnp.int32),
            pltpu.VMEM((1, ebs), jnp.int32),
            pltpu.VMEM((ebs, 128), _F32),
            pltpu.SemaphoreType.DMA,
        ],
        compiler_params=pltpu.CompilerParams(use_tc_tiling_on_sc=False),
    )
    def sage(h, esrc, edst, zz, o, acc, ssrc, sdst, idxd, rows, sem):
        cid = lax.axis_index("c")
        sid = lax.axis_index("s")
        e0 = sid * eps
        z0 = pl.multiple_of(sid * zps, zps)

        for rp in range(_NRNG // 2):
            lo = (cid * (_NRNG // 2) + rp) * rng
            pltpu.sync_copy(zz.at[pl.ds(z0, zps)], acc.at[pl.ds(z0, zps)])
            plsc.subcore_barrier()

            def scan_body(k, carry):
                soff = pl.multiple_of(e0 + k * scan, scan)
                pltpu.sync_copy(esrc.at[pl.ds(soff, scan)], ssrc)
                pltpu.sync_copy(edst.at[pl.ds(soff, scan)], sdst)

                def bat_body(bi, c2):
                    g = pltpu.async_copy(
                        h.at[ssrc.at[pl.ds(bi * ebs, ebs)]], rows, sem)
                    for v in range(ebs // _L):
                        d = sdst[pl.ds(bi * ebs + v * _L, _L)] - lo
                        ok = (d >= 0) & (d < rng)
                        idxd[0, pl.ds(v * _L, _L)] = jnp.where(ok, d, rng)
                    g.wait()
                    pltpu.sync_copy(rows, acc.at[idxd.at[0]], add=True)
                    return c2

                lax.fori_loop(0, nbat, bat_body, 0)
                return carry

            lax.fori_loop(0, nscan, scan_body, 0)
            plsc.subcore_barrier()

            @pl.when(sid < n_hi)
            def _():
                s0 = sid * cp_hi
                pltpu.sync_copy(acc.at[pl.ds(s0, cp_hi)],
                                o.at[pl.ds(lo + s0, cp_hi)])

            @pl.when(sid >= n_hi)
            def _():
                s0 = n_hi * cp_hi + (sid - n_hi) * cp_lo
                pltpu.sync_copy(acc.at[pl.ds(s0, cp_lo)],
                                o.at[pl.ds(lo + s0, cp_lo)])

            plsc.subcore_barrier()

    return sage


@functools.lru_cache(maxsize=None)
def _make_gather_sc(n_rows, k_pad, d):
    mesh = plsc.VectorSubcoreMesh(core_axis_name="c", subcore_axis_name="s")

    @functools.partial(
        pl.kernel,
        out_type=jax.ShapeDtypeStruct((k_pad, d), _F32),
        mesh=mesh,
        scratch_types=[
            pltpu.VMEM((_EB,), jnp.int32),
            pltpu.VMEM((_EB, d), _F32),
            pltpu.SemaphoreType.DMA,
        ],
        compiler_params=pltpu.CompilerParams(use_tc_tiling_on_sc=False),
    )
    def gather(tab, idx, o, idxv, rows, sem):
        cid = lax.axis_index("c")
        sid = lax.axis_index("s")
        wid = sid * 2 + cid

        @pl.when(wid < k_pad // _EB)
        def _():
            off = pl.multiple_of(wid * _EB, _EB)
            pltpu.sync_copy(idx.at[pl.ds(off, _EB)], idxv)
            pltpu.async_copy(tab.at[idxv], rows, sem).wait()
            pltpu.sync_copy(rows, o.at[pl.ds(off, _EB)])

    return gather


def _full_spec(arr):
    return pl.BlockSpec(arr.shape, lambda i: tuple(0 for _ in arr.shape))


def kernel(x, pre_x, edge_index, internal_edge_index, ppi_edge_index,
           num_entity, name_embeddings, desc_embeddings, batch_size, ko_mask,
           batch_ko_masks, W_name, b_name, W_desc, b_desc, W_omic, b_omic,
           W_fus, b_fus, W_pre, b_pre, W_ie_root, W_ie_nbr, b_ie, W_enc_root,
           W_enc_nbr, b_enc, Wg1, bg1, Wg2, bg2, Wreg, breg):
    n, f = x.shape
    ne = name_embeddings.shape[0]
    b_sz = batch_ko_masks.shape[0]
    kb = batch_ko_masks.shape[1]
    n_pad = ((n + 16 * _NSUB - 1) // (16 * _NSUB)) * (16 * _NSUB)
    e_int = internal_edge_index.astype(jnp.int32)
    e_ext = edge_index.astype(jnp.int32)
    n_edge = e_int.shape[1]

    ko = jnp.zeros((n, 1), _F32).at[ko_mask].set(1.0)
    wf1 = W_fus[0:128]
    wf2 = W_fus[128:256]
    wf3 = W_fus[256:383]
    r2 = lambda v: v.reshape(1, -1)

    ones_eb = jnp.ones((_EB,), _F32)
    zeros_1d = jnp.zeros((n_pad,), _F32)
    deg_int, deg_ext = _make_deg_sc(n_pad, n_edge)(
        e_int[1], e_ext[1], ones_eb, zeros_1d)
    deg_int2 = deg_int[:n].reshape(n, 1)
    deg_ext2 = deg_ext[:n].reshape(n, 1)

    grid1 = ne // _RB
    c_nd = pl.pallas_call(
        _a1_body,
        grid=(grid1,),
        in_specs=[
            pl.BlockSpec((_RB, 128), lambda i: (i, 0)),
            pl.BlockSpec((_RB, 128), lambda i: (i, 0)),
        ] + [_full_spec(a) for a in
             (W_name, r2(b_name), W_desc, r2(b_desc), wf1, wf2)],
        out_specs=pl.BlockSpec((_RB, f), lambda i: (i, 0)),
        out_shape=jax.ShapeDtypeStruct((ne, f), _F32),
    )(name_embeddings, desc_embeddings, W_name, r2(b_name), W_desc,
      r2(b_desc), wf1, wf2)

    grid2 = n // _RB
    nd_blocks = ne // _RB
    cx2, h1 = pl.pallas_call(
        _a2_body,
        grid=(grid2,),
        in_specs=[
            pl.BlockSpec((_RB, f), lambda i: (i, 0)),
            pl.BlockSpec((_RB, f), lambda i: (i % nd_blocks, 0)),
            pl.BlockSpec((_RB, 1), lambda i: (i, 0)),
        ] + [_full_spec(a) for a in
             (W_omic, r2(b_omic), wf3, r2(b_fus), W_ie_nbr)],
        out_specs=[pl.BlockSpec((_RB, 128), lambda i: (i, 0))] * 2,
        out_shape=[jax.ShapeDtypeStruct((n, 128), _F32)] * 2,
    )(x, c_nd, ko, W_omic, r2(b_omic), wf3, r2(b_fus), W_ie_nbr)

    sage = _make_sage_sc(n, n_edge)
    acc_rows = ((n // _NRNG + _NSUB) // _NSUB) * _NSUB
    zeros_acc = jnp.zeros((acc_rows, 128), _F32)
    agg1 = sage(h1, e_int[0], e_int[1], zeros_acc)

    z, h2 = pl.pallas_call(
        _b_body,
        grid=(grid2,),
        in_specs=[
            pl.BlockSpec((_RB, 128), lambda i: (i, 0)),
            pl.BlockSpec((_RB, 1), lambda i: (i, 0)),
            pl.BlockSpec((_RB, 128), lambda i: (i, 0)),
            pl.BlockSpec((_RB, f), lambda i: (i, 0)),
            pl.BlockSpec((_RB, f), lambda i: (i, 0)),
            pl.BlockSpec((_RB, 1), lambda i: (i, 0)),
        ] + [_full_spec(a) for a in
             (W_ie_root, r2(b_ie), W_pre, r2(b_pre), W_enc_nbr)],
        out_specs=[pl.BlockSpec((_RB, 128), lambda i: (i, 0))] * 2,
        out_shape=[jax.ShapeDtypeStruct((n, 128), _F32)] * 2,
    )(agg1, deg_int2, cx2, x, pre_x, ko, W_ie_root, r2(b_ie),
      W_pre, r2(b_pre), W_enc_nbr)

    agg2 = sage(h2, e_ext[0], e_ext[1], zeros_acc)

    z2 = pl.pallas_call(
        _c_body,
        grid=(grid2,),
        in_specs=[
            pl.BlockSpec((_RB, 128), lambda i: (i, 0)),
            pl.BlockSpec((_RB, 1), lambda i: (i, 0)),
            pl.BlockSpec((_RB, 128), lambda i: (i, 0)),
        ] + [_full_spec(a) for a in (W_enc_root, r2(b_enc))],
        out_specs=pl.BlockSpec((_RB, 128), lambda i: (i, 0)),
        out_shape=jax.ShapeDtypeStruct((n, 128), _F32),
    )(agg2, deg_ext2, z, W_enc_root, r2(b_enc))

    bb = jnp.minimum(jnp.arange(b_sz), batch_size - 1)
    idx = (batch_ko_masks[bb].astype(jnp.int32)
           + (bb * num_entity).astype(jnp.int32)[:, None]).reshape(-1)
    k_tot = b_sz * kb
    k_pad = ((k_tot + _EB - 1) // _EB) * _EB
    idx = jnp.pad(idx, (0, k_pad - k_tot))
    bz = _make_gather_sc(n, k_pad, 128)(z2, idx)[:k_tot]

    outd = pl.pallas_call(
        functools.partial(_d_body, nb=b_sz, kb=kb),
        out_shape=jax.ShapeDtypeStruct((b_sz, 1), _F32),
    )(bz, Wg1, r2(bg1), Wg2, r2(bg2), Wreg, r2(breg))
    return outd[:, 0]


# speedup vs baseline: 2.5980x; 1.2384x over previous
import functools

import jax
import jax.numpy as jnp
from jax import lax
from jax.experimental import pallas as pl
from jax.experimental.pallas import tpu as pltpu
from jax.experimental.pallas import tpu_sc as plsc

_F32 = jnp.float32
_EB = 128
_NSUB = 16
_NRNG = 4
_RB = 1000
_L = 16


def _lrelu(v):
    return jnp.where(v >= 0, v, 0.3 * v)


def _a1_body(ne_ref, de_ref, wn_ref, bn_ref, wd_ref, bd_ref, wf1_ref, wf2_ref,
             o_ref):
    a = _lrelu(jnp.dot(ne_ref[...], wn_ref[...],
                       preferred_element_type=_F32) + bn_ref[...])
    b = _lrelu(jnp.dot(de_ref[...], wd_ref[...],
                       preferred_element_type=_F32) + bd_ref[...])
    o_ref[...] = (jnp.dot(a, wf1_ref[...], preferred_element_type=_F32)
                  + jnp.dot(b, wf2_ref[...], preferred_element_type=_F32))


def _a2_body(x_ref, cnd_ref, ko_ref, womic_ref, bomic_ref, wf3_ref, bfus_ref,
             wn_ref, cx2_ref, h_ref):
    om = _lrelu(jnp.dot(x_ref[...], womic_ref[...],
                        preferred_element_type=_F32) + bomic_ref[...])
    cx = (cnd_ref[...] + jnp.dot(om, wf3_ref[...],
                                 preferred_element_type=_F32) + bfus_ref[...])
    cx2 = jnp.concatenate([cx, ko_ref[...]], axis=1)
    cx2_ref[...] = cx2
    h_ref[...] = jnp.dot(cx2, wn_ref[...], preferred_element_type=_F32)


def _b_body(agg_ref, deg_ref, cx2_ref, x_ref, px_ref, ko_ref,
            wroot_ref, bie_ref, wpre_ref, bpre_ref, wn2_ref,
            z_ref, h_ref):
    aggn = agg_ref[...] / jnp.maximum(deg_ref[...], 1.0)
    s1 = (jnp.dot(cx2_ref[...], wroot_ref[...], preferred_element_type=_F32)
          + aggn + bie_ref[...])
    x2 = jnp.concatenate([x_ref[...], ko_ref[...]], axis=1)
    px2 = jnp.concatenate([px_ref[...], ko_ref[...]], axis=1)
    z = (_lrelu(s1) + x2
         + jnp.dot(px2, wpre_ref[...], preferred_element_type=_F32)
         + bpre_ref[...])
    z_ref[...] = z
    h_ref[...] = jnp.dot(z, wn2_ref[...], preferred_element_type=_F32)


def _c_body(agg_ref, deg_ref, z_ref, wroot_ref, benc_ref, o_ref):
    aggn = agg_ref[...] / jnp.maximum(deg_ref[...], 1.0)
    o_ref[...] = _lrelu(
        jnp.dot(z_ref[...], wroot_ref[...], preferred_element_type=_F32)
        + aggn + benc_ref[...])


def _d_body(bz_ref, wg1_ref, bg1_ref, wg2_ref, bg2_ref, wreg_ref, breg_ref,
            o_ref, *, nb, kb):
    bz = bz_ref[...]
    t = jnp.tanh(jnp.dot(bz, wg1_ref[...], preferred_element_type=_F32)
                 + bg1_ref[...])
    s = jnp.dot(t, wg2_ref[...], preferred_element_type=_F32) + bg2_ref[...]
    rows = lax.broadcasted_iota(jnp.int32, (nb * kb, 1), 0)
    outs = []
    for b in range(nb):
        m = (rows >= b * kb) & (rows < (b + 1) * kb)
        sm = jnp.where(m, s, -jnp.inf)
        mx = jnp.max(sm)
        e = jnp.where(m, jnp.exp(s - mx), 0.0)
        w = e / jnp.sum(e)
        rd = jnp.sum(w * bz, axis=0, keepdims=True)
        outs.append(jnp.dot(rd, wreg_ref[...], preferred_element_type=_F32)
                    + breg_ref[...])
    o_ref[...] = jnp.concatenate(outs, axis=0)


def _batch_split(n_batches, sid):
    base, rem = divmod(n_batches, _NSUB)
    nb = jnp.where(sid < rem, base + 1, base)
    b0 = sid * base + jnp.minimum(sid, rem)
    return b0, nb


@functools.lru_cache(maxsize=None)
def _make_deg_sc(n_pad, n_edge):
    nb_total = n_edge // _EB
    rps = n_pad // _NSUB
    mesh = plsc.VectorSubcoreMesh(core_axis_name="c", subcore_axis_name="s")

    @functools.partial(
        pl.kernel,
        out_type=[jax.ShapeDtypeStruct((n_pad,), _F32) for _ in range(2)],
        mesh=mesh,
        scratch_types=[
            pltpu.VMEM_SHARED((n_pad,), _F32),
            pltpu.VMEM((1, _EB), jnp.int32),
            pltpu.VMEM((_EB,), _F32),
        ],
        compiler_params=pltpu.CompilerParams(use_tc_tiling_on_sc=False),
    )
    def deg(dst_a, dst_b, ones, zz, oa, ob, acc, idxd, onesv):
        cid = lax.axis_index("c")
        sid = lax.axis_index("s")
        b0, nb = _batch_split(nb_total, sid)
        r0 = pl.multiple_of(sid * rps, rps)
        pltpu.sync_copy(ones, onesv)

        def one_pass(dst, o_hbm):
            pltpu.sync_copy(zz.at[pl.ds(r0, rps)], acc.at[pl.ds(r0, rps)])
            plsc.subcore_barrier()

            def body(i, carry):
                off = pl.multiple_of((b0 + i) * _EB, _EB)
                pltpu.sync_copy(dst.at[pl.ds(off, _EB)], idxd.at[0])
                pltpu.sync_copy(onesv, acc.at[idxd.at[0]], add=True)
                return carry

            lax.fori_loop(0, nb, body, 0)
            plsc.subcore_barrier()
            pltpu.sync_copy(acc.at[pl.ds(r0, rps)], o_hbm.at[pl.ds(r0, rps)])

        @pl.when(cid == 0)
        def _():
            one_pass(dst_a, oa)

        @pl.when(cid == 1)
        def _():
            one_pass(dst_b, ob)

    return deg


@functools.lru_cache(maxsize=None)
def _make_sage_sc(n, n_edge):
    ebs = 80
    nb_total = n_edge // ebs
    rng = n // _NRNG
    acc_rows = ((rng + 1 + _NSUB - 1) // _NSUB) * _NSUB
    zps = acc_rows // _NSUB
    cp_hi = -(-rng // _NSUB)
    n_hi = rng - _NSUB * (rng // _NSUB)
    cp_lo = rng // _NSUB
    mesh = plsc.VectorSubcoreMesh(core_axis_name="c", subcore_axis_name="s")

    scan = 2000
    ebs = 80
    eps = n_edge // _NSUB
    nscan = eps // scan
    nbat = scan // ebs

    @functools.partial(
        pl.kernel,
        out_type=jax.ShapeDtypeStruct((n, 128), _F32),
        mesh=mesh,
        scratch_types=[
            pltpu.VMEM_SHARED((acc_rows, 128), _F32),
            pltpu.VMEM((scan,), jnp.int32),
            pltpu.VMEM((scan,), jnp.int32),
            pltpu.VMEM((2, ebs), jnp.int32),
            pltpu.VMEM((2, ebs, 128), _F32),
            pltpu.SemaphoreType.DMA,
        ],
        compiler_params=pltpu.CompilerParams(use_tc_tiling_on_sc=False),
    )
    def sage(h, esrc, edst, zz, o, acc, ssrc, sdst, idxd, rows, sem):
        cid = lax.axis_index("c")
        sid = lax.axis_index("s")
        e0 = sid * eps
        z0 = pl.multiple_of(sid * zps, zps)
        ntot = nscan * nbat

        for rp in range(_NRNG // 2):
            lo = (cid * (_NRNG // 2) + rp) * rng
            pltpu.sync_copy(zz.at[pl.ds(z0, zps)], acc.at[pl.ds(z0, zps)])
            plsc.subcore_barrier()

            def bat_body(i, carry):
                bt = lax.rem(i, 2)
                bi = lax.rem(i, nbat)

                @pl.when(bi == 0)
                def _():
                    soff = pl.multiple_of(e0 + (i // nbat) * scan, scan)
                    pltpu.sync_copy(esrc.at[pl.ds(soff, scan)], ssrc)
                    pltpu.sync_copy(edst.at[pl.ds(soff, scan)], sdst)

                pltpu.async_copy(
                    h.at[ssrc.at[pl.ds(bi * ebs, ebs)]], rows.at[bt], sem)
                for v in range(ebs // _L):
                    d = sdst[pl.ds(bi * ebs + v * _L, _L)] - lo
                    ok = (d >= 0) & (d < rng)
                    idxd[bt, pl.ds(v * _L, _L)] = jnp.where(ok, d, rng)

                @pl.when(i > 0)
                def _():
                    ot = lax.rem(i + 1, 2)
                    pltpu.make_async_copy(
                        h.at[ssrc.at[pl.ds(bi * ebs, ebs)]], rows.at[ot],
                        sem).wait()
                    pltpu.sync_copy(rows.at[ot], acc.at[idxd.at[ot]],
                                    add=True)
                return carry

            lax.fori_loop(0, ntot, bat_body, 0)
            lt = lax.rem(ntot + 1, 2)
            pltpu.make_async_copy(
                h.at[ssrc.at[pl.ds((nbat - 1) * ebs, ebs)]], rows.at[lt],
                sem).wait()
            pltpu.sync_copy(rows.at[lt], acc.at[idxd.at[lt]], add=True)
            plsc.subcore_barrier()

            @pl.when(sid < n_hi)
            def _():
                s0 = sid * cp_hi
                pltpu.sync_copy(acc.at[pl.ds(s0, cp_hi)],
                                o.at[pl.ds(lo + s0, cp_hi)])

            @pl.when(sid >= n_hi)
            def _():
                s0 = n_hi * cp_hi + (sid - n_hi) * cp_lo
                pltpu.sync_copy(acc.at[pl.ds(s0, cp_lo)],
                                o.at[pl.ds(lo + s0, cp_lo)])

            plsc.subcore_barrier()

    return sage


@functools.lru_cache(maxsize=None)
def _make_gather_sc(n_rows, k_pad, d):
    mesh = plsc.VectorSubcoreMesh(core_axis_name="c", subcore_axis_name="s")

    @functools.partial(
        pl.kernel,
        out_type=jax.ShapeDtypeStruct((k_pad, d), _F32),
        mesh=mesh,
        scratch_types=[
            pltpu.VMEM((_EB,), jnp.int32),
            pltpu.VMEM((_EB, d), _F32),
            pltpu.SemaphoreType.DMA,
        ],
        compiler_params=pltpu.CompilerParams(use_tc_tiling_on_sc=False),
    )
    def gather(tab, idx, o, idxv, rows, sem):
        cid = lax.axis_index("c")
        sid = lax.axis_index("s")
        wid = sid * 2 + cid

        @pl.when(wid < k_pad // _EB)
        def _():
            off = pl.multiple_of(wid * _EB, _EB)
            pltpu.sync_copy(idx.at[pl.ds(off, _EB)], idxv)
            pltpu.async_copy(tab.at[idxv], rows, sem).wait()
            pltpu.sync_copy(rows, o.at[pl.ds(off, _EB)])

    return gather


def _full_spec(arr):
    return pl.BlockSpec(arr.shape, lambda i: tuple(0 for _ in arr.shape))


def kernel(x, pre_x, edge_index, internal_edge_index, ppi_edge_index,
           num_entity, name_embeddings, desc_embeddings, batch_size, ko_mask,
           batch_ko_masks, W_name, b_name, W_desc, b_desc, W_omic, b_omic,
           W_fus, b_fus, W_pre, b_pre, W_ie_root, W_ie_nbr, b_ie, W_enc_root,
           W_enc_nbr, b_enc, Wg1, bg1, Wg2, bg2, Wreg, breg):
    n, f = x.shape
    ne = name_embeddings.shape[0]
    b_sz = batch_ko_masks.shape[0]
    kb = batch_ko_masks.shape[1]
    n_pad = ((n + 16 * _NSUB - 1) // (16 * _NSUB)) * (16 * _NSUB)
    e_int = internal_edge_index.astype(jnp.int32)
    e_ext = edge_index.astype(jnp.int32)
    n_edge = e_int.shape[1]

    ko = jnp.zeros((n, 1), _F32).at[ko_mask].set(1.0)
    wf1 = W_fus[0:128]
    wf2 = W_fus[128:256]
    wf3 = W_fus[256:383]
    r2 = lambda v: v.reshape(1, -1)

    ones_eb = jnp.ones((_EB,), _F32)
    zeros_1d = jnp.zeros((n_pad,), _F32)
    deg_int, deg_ext = _make_deg_sc(n_pad, n_edge)(
        e_int[1], e_ext[1], ones_eb, zeros_1d)
    deg_int2 = deg_int[:n].reshape(n, 1)
    deg_ext2 = deg_ext[:n].reshape(n, 1)

    grid1 = ne // _RB
    c_nd = pl.pallas_call(
        _a1_body,
        grid=(grid1,),
        in_specs=[
            pl.BlockSpec((_RB, 128), lambda i: (i, 0)),
            pl.BlockSpec((_RB, 128), lambda i: (i, 0)),
        ] + [_full_spec(a) for a in
             (W_name, r2(b_name), W_desc, r2(b_desc), wf1, wf2)],
        out_specs=pl.BlockSpec((_RB, f), lambda i: (i, 0)),
        out_shape=jax.ShapeDtypeStruct((ne, f), _F32),
    )(name_embeddings, desc_embeddings, W_name, r2(b_name), W_desc,
      r2(b_desc), wf1, wf2)

    grid2 = n // _RB
    nd_blocks = ne // _RB
    cx2, h1 = pl.pallas_call(
        _a2_body,
        grid=(grid2,),
        in_specs=[
            pl.BlockSpec((_RB, f), lambda i: (i, 0)),
            pl.BlockSpec((_RB, f), lambda i: (i % nd_blocks, 0)),
            pl.BlockSpec((_RB, 1), lambda i: (i, 0)),
        ] + [_full_spec(a) for a in
             (W_omic, r2(b_omic), wf3, r2(b_fus), W_ie_nbr)],
        out_specs=[pl.BlockSpec((_RB, 128), lambda i: (i, 0))] * 2,
        out_shape=[jax.ShapeDtypeStruct((n, 128), _F32)] * 2,
    )(x, c_nd, ko, W_omic, r2(b_omic), wf3, r2(b_fus), W_ie_nbr)

    sage = _make_sage_sc(n, n_edge)
    acc_rows = ((n // _NRNG + _NSUB) // _NSUB) * _NSUB
    zeros_acc = jnp.zeros((acc_rows, 128), _F32)
    agg1 = sage(h1, e_int[0], e_int[1], zeros_acc)

    z, h2 = pl.pallas_call(
        _b_body,
        grid=(grid2,),
        in_specs=[
            pl.BlockSpec((_RB, 128), lambda i: (i, 0)),
            pl.BlockSpec((_RB, 1), lambda i: (i, 0)),
            pl.BlockSpec((_RB, 128), lambda i: (i, 0)),
            pl.BlockSpec((_RB, f), lambda i: (i, 0)),
            pl.BlockSpec((_RB, f), lambda i: (i, 0)),
            pl.BlockSpec((_RB, 1), lambda i: (i, 0)),
        ] + [_full_spec(a) for a in
             (W_ie_root, r2(b_ie), W_pre, r2(b_pre), W_enc_nbr)],
        out_specs=[pl.BlockSpec((_RB, 128), lambda i: (i, 0))] * 2,
        out_shape=[jax.ShapeDtypeStruct((n, 128), _F32)] * 2,
    )(agg1, deg_int2, cx2, x, pre_x, ko, W_ie_root, r2(b_ie),
      W_pre, r2(b_pre), W_enc_nbr)

    agg2 = sage(h2, e_ext[0], e_ext[1], zeros_acc)

    z2 = pl.pallas_call(
        _c_body,
        grid=(grid2,),
        in_specs=[
            pl.BlockSpec((_RB, 128), lambda i: (i, 0)),
            pl.BlockSpec((_RB, 1), lambda i: (i, 0)),
            pl.BlockSpec((_RB, 128), lambda i: (i, 0)),
        ] + [_full_spec(a) for a in (W_enc_root, r2(b_enc))],
        out_specs=pl.BlockSpec((_RB, 128), lambda i: (i, 0)),
        out_shape=jax.ShapeDtypeStruct((n, 128), _F32),
    )(agg2, deg_ext2, z, W_enc_root, r2(b_enc))

    bb = jnp.minimum(jnp.arange(b_sz), batch_size - 1)
    idx = (batch_ko_masks[bb].astype(jnp.int32)
           + (bb * num_entity).astype(jnp.int32)[:, None]).reshape(-1)
    k_tot = b_sz * kb
    k_pad = ((k_tot + _EB - 1) // _EB) * _EB
    idx = jnp.pad(idx, (0, k_pad - k_tot))
    bz = _make_gather_sc(n, k_pad, 128)(z2, idx)[:k_tot]

    outd = pl.pallas_call(
        functools.partial(_d_body, nb=b_sz, kb=kb),
        out_shape=jax.ShapeDtypeStruct((b_sz, 1), _F32),
    )(bz, Wg1, r2(bg1), Wg2, r2(bg2), Wreg, r2(breg))
    return outd[:, 0]
